# baseline (device time: 333087 ns/iter reference)
import jax
import jax.numpy as jnp
from jax import lax
from jax.experimental import pallas as pl
from jax.experimental.pallas import tpu as pltpu

N_DEV = 8
S_LOC = 512
D = 1024
H_LOC = 8
DH = 128
S_GLOB = N_DEV * S_LOC
SCALE = 0.08838834764831843
LOG2E = 1.4426950408889634
R_HOPS = 4
L_HOPS = 3


def kernel(x, Wq, Wo, Wk, Wv):
    def body(
        x_ref, wq_ref, wo_ref, wk_ref, wv_ref, out_ref,
        xg_ref, k_ref, v_ref, rcv_ref, snd_ref,
        agr_ssem, agr_rsem, agl_ssem, agl_rsem, rs_ssem, rs_rsem,
    ):
        p = lax.axis_index("i")
        left = (p - 1) % N_DEV
        right = (p + 1) % N_DEV

        bar = pltpu.get_barrier_semaphore()
        for nbr in (left, right):
            pl.semaphore_signal(
                bar, inc=1, device_id=(nbr,),
                device_id_type=pl.DeviceIdType.MESH,
            )
        pl.semaphore_wait(bar, 2)

        xg_ref[pl.ds(p, 1)] = x_ref[...]

        def kv_chunk(c):
            xc = xg_ref[pl.ds(c, 1)][0]
            k_ref[pl.ds(c * S_LOC, S_LOC), :] = jnp.dot(
                xc, wk_ref[...], preferred_element_type=jnp.float32
            ).astype(jnp.bfloat16)
            v_ref[pl.ds(c * S_LOC, S_LOC), :] = jnp.dot(
                xc, wv_ref[...], preferred_element_type=jnp.float32
            ).astype(jnp.bfloat16)

        for h in range(R_HOPS):
            sr = (p - h) % N_DEV
            r_rdma = pltpu.make_async_remote_copy(
                src_ref=xg_ref.at[sr],
                dst_ref=xg_ref.at[sr],
                send_sem=agr_ssem.at[h],
                recv_sem=agr_rsem.at[h],
                device_id=(right,),
                device_id_type=pl.DeviceIdType.MESH,
            )
            r_rdma.start()
            l_rdma = None
            if h < L_HOPS:
                sl = (p + h) % N_DEV
                l_rdma = pltpu.make_async_remote_copy(
                    src_ref=xg_ref.at[sl],
                    dst_ref=xg_ref.at[sl],
                    send_sem=agl_ssem.at[h],
                    recv_sem=agl_rsem.at[h],
                    device_id=(left,),
                    device_id_type=pl.DeviceIdType.MESH,
                )
                l_rdma.start()
            if h == 0:
                kv_chunk(p)
            else:
                kv_chunk((p - h) % N_DEV)
                kv_chunk((p + h) % N_DEV)
            r_rdma.wait()
            if l_rdma is not None:
                l_rdma.wait()
        kv_chunk((p + R_HOPS) % N_DEV)

        def attn_step(t, carry):
            c = (p - 1 - t) % N_DEV
            xq = xg_ref[pl.ds(c, 1)][0]
            qall = (
                jnp.dot(xq, wq_ref[...], preferred_element_type=jnp.float32)
                * (SCALE * LOG2E)
            ).astype(jnp.bfloat16)
            ones_m = jnp.ones((S_GLOB, DH), jnp.bfloat16)
            os = []
            for h in range(H_LOC):
                col = slice(h * DH, (h + 1) * DH)
                s = lax.dot_general(
                    qall[:, col], k_ref[:, col], (((1,), (1,)), ((), ())),
                    preferred_element_type=jnp.float32,
                ).astype(jnp.bfloat16)
                pexp = jnp.exp2(s)
                l = jnp.dot(
                    pexp, ones_m, preferred_element_type=jnp.float32
                )[:, :1]
                o = jnp.dot(
                    pexp, v_ref[:, col], preferred_element_type=jnp.float32
                )
                os.append((o / l).astype(jnp.bfloat16))
            o_all = jnp.concatenate(os, axis=1)
            acc = jnp.dot(
                o_all, wo_ref[...], preferred_element_type=jnp.float32
            )
            tm1 = jnp.maximum(t - 1, 0)

            @pl.when(t > 0)
            def _retire_prev():
                prev_rdma = pltpu.make_async_remote_copy(
                    src_ref=snd_ref,
                    dst_ref=rcv_ref.at[tm1],
                    send_sem=rs_ssem.at[tm1],
                    recv_sem=rs_rsem.at[tm1],
                    device_id=(right,),
                    device_id_type=pl.DeviceIdType.MESH,
                )
                prev_rdma.wait_send()
                prev_rdma.wait_recv()

            prev = rcv_ref[pl.ds(tm1, 1)][0].astype(jnp.float32)
            acc = acc + jnp.where(t > 0, prev, jnp.float32(0.0))

            @pl.when(t < N_DEV - 1)
            def _send():
                snd_ref[...] = acc.astype(jnp.bfloat16)
                rdma = pltpu.make_async_remote_copy(
                    src_ref=snd_ref,
                    dst_ref=rcv_ref.at[t],
                    send_sem=rs_ssem.at[t],
                    recv_sem=rs_rsem.at[t],
                    device_id=(right,),
                    device_id_type=pl.DeviceIdType.MESH,
                )
                rdma.start()

            @pl.when(t == N_DEV - 1)
            def _finish():
                out_ref[...] = acc[None]

            return carry

        lax.fori_loop(0, N_DEV, attn_step, 0)

    f = pl.pallas_call(
        body,
        out_shape=jax.ShapeDtypeStruct((1, S_LOC, D), jnp.float32),
        in_specs=[pl.BlockSpec(memory_space=pltpu.VMEM)] * 5,
        out_specs=pl.BlockSpec(memory_space=pltpu.VMEM),
        scratch_shapes=[
            pltpu.VMEM((N_DEV, S_LOC, D), jnp.bfloat16),
            pltpu.VMEM((S_GLOB, D), jnp.bfloat16),
            pltpu.VMEM((S_GLOB, D), jnp.bfloat16),
            pltpu.VMEM((N_DEV - 1, S_LOC, D), jnp.bfloat16),
            pltpu.VMEM((S_LOC, D), jnp.bfloat16),
            pltpu.SemaphoreType.DMA((R_HOPS,)),
            pltpu.SemaphoreType.DMA((R_HOPS,)),
            pltpu.SemaphoreType.DMA((L_HOPS,)),
            pltpu.SemaphoreType.DMA((L_HOPS,)),
            pltpu.SemaphoreType.DMA((N_DEV - 1,)),
            pltpu.SemaphoreType.DMA((N_DEV - 1,)),
        ],
        compiler_params=pltpu.CompilerParams(
            collective_id=0, vmem_limit_bytes=63 * 1024 * 1024
        ),
    )
    return f(
        x.astype(jnp.bfloat16),
        Wq.astype(jnp.bfloat16),
        Wo.astype(jnp.bfloat16),
        Wk.astype(jnp.bfloat16),
        Wv.astype(jnp.bfloat16),
    )


# device time: 286351 ns/iter; 1.1632x vs baseline; 1.1632x over previous
import jax
import jax.numpy as jnp
from jax import lax
from jax.experimental import pallas as pl
from jax.experimental.pallas import tpu as pltpu

N_DEV = 8
S_LOC = 512
D = 1024
H_LOC = 8
DH = 128
S_GLOB = N_DEV * S_LOC
SCALE = 0.08838834764831843
LOG2E = 1.4426950408889634
R_HOPS = 4
L_HOPS = 3


def kernel(x, Wq, Wo, Wk, Wv):
    def body(
        x_ref, wq_ref, wo_ref, wk_ref, wv_ref, out_ref,
        xg_ref, k_ref, v_ref, rcv_ref, snd_ref,
        agr_ssem, agr_rsem, agl_ssem, agl_rsem, rs_ssem, rs_rsem,
    ):
        p = lax.axis_index("i")
        left = (p - 1) % N_DEV
        right = (p + 1) % N_DEV

        bar = pltpu.get_barrier_semaphore()
        for nbr in (left, right):
            pl.semaphore_signal(
                bar, inc=1, device_id=(nbr,),
                device_id_type=pl.DeviceIdType.MESH,
            )
        pl.semaphore_wait(bar, 2)

        xg_ref[pl.ds(p, 1)] = x_ref[...]

        def kv_chunk(c):
            xc = xg_ref[pl.ds(c, 1)][0]
            k_ref[pl.ds(c * S_LOC, S_LOC), :] = jnp.dot(
                xc, wk_ref[...], preferred_element_type=jnp.float32
            ).astype(jnp.bfloat16)
            v_ref[pl.ds(c * S_LOC, S_LOC), :] = jnp.dot(
                xc, wv_ref[...], preferred_element_type=jnp.float32
            ).astype(jnp.bfloat16)

        for h in range(R_HOPS):
            sr = (p - h) % N_DEV
            r_rdma = pltpu.make_async_remote_copy(
                src_ref=xg_ref.at[sr],
                dst_ref=xg_ref.at[sr],
                send_sem=agr_ssem.at[h],
                recv_sem=agr_rsem.at[h],
                device_id=(right,),
                device_id_type=pl.DeviceIdType.MESH,
            )
            r_rdma.start()
            l_rdma = None
            if h < L_HOPS:
                sl = (p + h) % N_DEV
                l_rdma = pltpu.make_async_remote_copy(
                    src_ref=xg_ref.at[sl],
                    dst_ref=xg_ref.at[sl],
                    send_sem=agl_ssem.at[h],
                    recv_sem=agl_rsem.at[h],
                    device_id=(left,),
                    device_id_type=pl.DeviceIdType.MESH,
                )
                l_rdma.start()
            if h == 0:
                kv_chunk(p)
            else:
                kv_chunk((p - h) % N_DEV)
                kv_chunk((p + h) % N_DEV)
            r_rdma.wait()
            if l_rdma is not None:
                l_rdma.wait()
        kv_chunk((p + R_HOPS) % N_DEV)

        def attn_step(t, carry):
            c = (p - 1 - t) % N_DEV
            xq = xg_ref[pl.ds(c, 1)][0]
            qall = (
                jnp.dot(xq, wq_ref[...], preferred_element_type=jnp.float32)
                * (SCALE * LOG2E)
            ).astype(jnp.bfloat16)
            os = []
            for h in range(H_LOC):
                col = slice(h * DH, (h + 1) * DH)
                s = lax.dot_general(
                    qall[:, col], k_ref[:, col], (((1,), (1,)), ((), ())),
                    preferred_element_type=jnp.float32,
                ).astype(jnp.bfloat16)
                pexp = jnp.exp2(s)
                l = jnp.sum(pexp, axis=1, keepdims=True, dtype=jnp.float32)
                o = jnp.dot(
                    pexp, v_ref[:, col], preferred_element_type=jnp.float32
                )
                os.append((o / l).astype(jnp.bfloat16))
            o_all = jnp.concatenate(os, axis=1)
            acc = jnp.dot(
                o_all, wo_ref[...], preferred_element_type=jnp.float32
            )
            tm1 = jnp.maximum(t - 1, 0)

            @pl.when(t > 0)
            def _retire_prev():
                prev_rdma = pltpu.make_async_remote_copy(
                    src_ref=snd_ref,
                    dst_ref=rcv_ref.at[tm1],
                    send_sem=rs_ssem.at[tm1],
                    recv_sem=rs_rsem.at[tm1],
                    device_id=(right,),
                    device_id_type=pl.DeviceIdType.MESH,
                )
                prev_rdma.wait_send()
                prev_rdma.wait_recv()

            prev = rcv_ref[pl.ds(tm1, 1)][0].astype(jnp.float32)
            acc = acc + jnp.where(t > 0, prev, jnp.float32(0.0))

            @pl.when(t < N_DEV - 1)
            def _send():
                snd_ref[...] = acc.astype(jnp.bfloat16)
                rdma = pltpu.make_async_remote_copy(
                    src_ref=snd_ref,
                    dst_ref=rcv_ref.at[t],
                    send_sem=rs_ssem.at[t],
                    recv_sem=rs_rsem.at[t],
                    device_id=(right,),
                    device_id_type=pl.DeviceIdType.MESH,
                )
                rdma.start()

            @pl.when(t == N_DEV - 1)
            def _finish():
                out_ref[...] = acc[None]

            return carry

        lax.fori_loop(0, N_DEV, attn_step, 0)

    f = pl.pallas_call(
        body,
        out_shape=jax.ShapeDtypeStruct((1, S_LOC, D), jnp.float32),
        in_specs=[pl.BlockSpec(memory_space=pltpu.VMEM)] * 5,
        out_specs=pl.BlockSpec(memory_space=pltpu.VMEM),
        scratch_shapes=[
            pltpu.VMEM((N_DEV, S_LOC, D), jnp.bfloat16),
            pltpu.VMEM((S_GLOB, D), jnp.bfloat16),
            pltpu.VMEM((S_GLOB, D), jnp.bfloat16),
            pltpu.VMEM((N_DEV - 1, S_LOC, D), jnp.bfloat16),
            pltpu.VMEM((S_LOC, D), jnp.bfloat16),
            pltpu.SemaphoreType.DMA((R_HOPS,)),
            pltpu.SemaphoreType.DMA((R_HOPS,)),
            pltpu.SemaphoreType.DMA((L_HOPS,)),
            pltpu.SemaphoreType.DMA((L_HOPS,)),
            pltpu.SemaphoreType.DMA((N_DEV - 1,)),
            pltpu.SemaphoreType.DMA((N_DEV - 1,)),
        ],
        compiler_params=pltpu.CompilerParams(
            collective_id=0, vmem_limit_bytes=63 * 1024 * 1024
        ),
    )
    return f(
        x.astype(jnp.bfloat16),
        Wq.astype(jnp.bfloat16),
        Wo.astype(jnp.bfloat16),
        Wk.astype(jnp.bfloat16),
        Wv.astype(jnp.bfloat16),
    )


# device time: 259271 ns/iter; 1.2847x vs baseline; 1.1044x over previous
import jax
import jax.numpy as jnp
from jax import lax
from jax.experimental import pallas as pl
from jax.experimental.pallas import tpu as pltpu

N_DEV = 8
S_LOC = 512
D = 1024
H_LOC = 8
DH = 128
S_GLOB = N_DEV * S_LOC
SCALE = 0.08838834764831843
LOG2E = 1.4426950408889634
R_HOPS = 4
L_HOPS = 3


def kernel(x, Wq, Wo, Wk, Wv):
    def body(
        x_ref, wq_ref, wo_ref, wk_ref, wv_ref, out_ref,
        xg_ref, k_ref, v_ref, rcv_ref, snd_ref,
        agr_ssem, agr_rsem, agl_ssem, agl_rsem, rs_ssem, rs_rsem,
    ):
        p = lax.axis_index("i")
        left = (p - 1) % N_DEV
        right = (p + 1) % N_DEV

        bar = pltpu.get_barrier_semaphore()
        for nbr in (left, right):
            pl.semaphore_signal(
                bar, inc=1, device_id=(nbr,),
                device_id_type=pl.DeviceIdType.MESH,
            )
        pl.semaphore_wait(bar, 2)

        xg_ref[pl.ds(p, 1)] = x_ref[...]

        def kv_chunk(c):
            xc = xg_ref[pl.ds(c, 1)][0]
            k_ref[pl.ds(c * S_LOC, S_LOC), :] = jnp.dot(
                xc, wk_ref[...], preferred_element_type=jnp.float32
            ).astype(jnp.bfloat16)
            v_ref[pl.ds(c * S_LOC, S_LOC), :] = jnp.dot(
                xc, wv_ref[...], preferred_element_type=jnp.float32
            ).astype(jnp.bfloat16)

        for h in range(R_HOPS):
            sr = (p - h) % N_DEV
            r_rdma = pltpu.make_async_remote_copy(
                src_ref=xg_ref.at[sr],
                dst_ref=xg_ref.at[sr],
                send_sem=agr_ssem.at[h],
                recv_sem=agr_rsem.at[h],
                device_id=(right,),
                device_id_type=pl.DeviceIdType.MESH,
            )
            r_rdma.start()
            l_rdma = None
            if h < L_HOPS:
                sl = (p + h) % N_DEV
                l_rdma = pltpu.make_async_remote_copy(
                    src_ref=xg_ref.at[sl],
                    dst_ref=xg_ref.at[sl],
                    send_sem=agl_ssem.at[h],
                    recv_sem=agl_rsem.at[h],
                    device_id=(left,),
                    device_id_type=pl.DeviceIdType.MESH,
                )
                l_rdma.start()
            if h == 0:
                kv_chunk(p)
            else:
                kv_chunk((p - h) % N_DEV)
                kv_chunk((p + h) % N_DEV)
            r_rdma.wait()
            if l_rdma is not None:
                l_rdma.wait()
        kv_chunk((p + R_HOPS) % N_DEV)

        def attn_step(t, carry):
            c = (p - 1 - t) % N_DEV
            xq = xg_ref[pl.ds(c, 1)][0]
            qall = (
                jnp.dot(xq, wq_ref[...], preferred_element_type=jnp.float32)
                * SCALE
            ).astype(jnp.bfloat16)

            def s_block(h):
                col = slice(h * DH, (h + 1) * DH)
                return lax.dot_general(
                    qall[:, col], k_ref[:, col], (((1,), (1,)), ((), ())),
                    preferred_element_type=jnp.float32,
                ).astype(jnp.bfloat16)

            os = []
            s_cur = s_block(0)
            for h in range(H_LOC):
                col = slice(h * DH, (h + 1) * DH)
                s_nxt = s_block(h + 1) if h + 1 < H_LOC else None
                pexp = jnp.exp(s_cur)
                l = jnp.sum(pexp, axis=1, keepdims=True, dtype=jnp.float32)
                o = jnp.dot(
                    pexp, v_ref[:, col], preferred_element_type=jnp.float32
                )
                os.append((o / l).astype(jnp.bfloat16))
                s_cur = s_nxt
            o_all = jnp.concatenate(os, axis=1)
            acc = jnp.dot(
                o_all, wo_ref[...], preferred_element_type=jnp.float32
            )
            tm1 = jnp.maximum(t - 1, 0)

            @pl.when(t > 0)
            def _retire_prev():
                prev_rdma = pltpu.make_async_remote_copy(
                    src_ref=snd_ref,
                    dst_ref=rcv_ref.at[tm1],
                    send_sem=rs_ssem.at[tm1],
                    recv_sem=rs_rsem.at[tm1],
                    device_id=(right,),
                    device_id_type=pl.DeviceIdType.MESH,
                )
                prev_rdma.wait_send()
                prev_rdma.wait_recv()

            prev = rcv_ref[pl.ds(tm1, 1)][0].astype(jnp.float32)
            acc = acc + jnp.where(t > 0, prev, jnp.float32(0.0))

            @pl.when(t < N_DEV - 1)
            def _send():
                snd_ref[...] = acc.astype(jnp.bfloat16)
                rdma = pltpu.make_async_remote_copy(
                    src_ref=snd_ref,
                    dst_ref=rcv_ref.at[t],
                    send_sem=rs_ssem.at[t],
                    recv_sem=rs_rsem.at[t],
                    device_id=(right,),
                    device_id_type=pl.DeviceIdType.MESH,
                )
                rdma.start()

            @pl.when(t == N_DEV - 1)
            def _finish():
                out_ref[...] = acc[None]

            return carry

        lax.fori_loop(0, N_DEV, attn_step, 0)

    f = pl.pallas_call(
        body,
        out_shape=jax.ShapeDtypeStruct((1, S_LOC, D), jnp.float32),
        in_specs=[pl.BlockSpec(memory_space=pltpu.VMEM)] * 5,
        out_specs=pl.BlockSpec(memory_space=pltpu.VMEM),
        scratch_shapes=[
            pltpu.VMEM((N_DEV, S_LOC, D), jnp.bfloat16),
            pltpu.VMEM((S_GLOB, D), jnp.bfloat16),
            pltpu.VMEM((S_GLOB, D), jnp.bfloat16),
            pltpu.VMEM((N_DEV - 1, S_LOC, D), jnp.bfloat16),
            pltpu.VMEM((S_LOC, D), jnp.bfloat16),
            pltpu.SemaphoreType.DMA((R_HOPS,)),
            pltpu.SemaphoreType.DMA((R_HOPS,)),
            pltpu.SemaphoreType.DMA((L_HOPS,)),
            pltpu.SemaphoreType.DMA((L_HOPS,)),
            pltpu.SemaphoreType.DMA((N_DEV - 1,)),
            pltpu.SemaphoreType.DMA((N_DEV - 1,)),
        ],
        compiler_params=pltpu.CompilerParams(
            collective_id=0, vmem_limit_bytes=63 * 1024 * 1024
        ),
    )
    return f(
        x.astype(jnp.bfloat16),
        Wq.astype(jnp.bfloat16),
        Wo.astype(jnp.bfloat16),
        Wk.astype(jnp.bfloat16),
        Wv.astype(jnp.bfloat16),
    )
